# baseline (speedup 1.0000x reference)
import jax
import jax.numpy as jnp
from jax.experimental import pallas as pl
from jax.experimental.pallas import tpu as pltpu

N_BUCKET = 64
EPB = 8
NKEYS = N_BUCKET * EPB


def _router_block(h_ref, b_ref, keys_ref, gid_ref, w_ref, kn_ref):
    @pl.when(pl.program_id(0) == 0)
    def _normalize_keys():
        keys = keys_ref[...]
        norm = jnp.sqrt(jnp.sum(keys * keys, axis=1, keepdims=True))
        kn_ref[...] = keys * (1.0 / jnp.maximum(norm, 1e-12))

    h = h_ref[...]
    rh = 1.0 / jnp.maximum(jnp.sqrt(jnp.sum(h * h, axis=1, keepdims=True)), 1e-12)

    scores = jax.lax.dot_general(
        h, kn_ref[...], (((1,), (1,)), ((), ())),
        preferred_element_type=jnp.float32,
        precision=jax.lax.Precision.DEFAULT,
    )

    Tb = h.shape[0]
    b = b_ref[...]
    g_iota = jax.lax.broadcasted_iota(jnp.int32, (Tb, N_BUCKET), 1)
    oh = jnp.where(g_iota == b, 1.0, 0.0)

    sel = jnp.concatenate(
        [
            jnp.sum(scores[:, j * N_BUCKET:(j + 1) * N_BUCKET] * oh,
                    axis=1, keepdims=True)
            for j in range(EPB)
        ],
        axis=1,
    ) * rh

    lane8 = jax.lax.broadcasted_iota(jnp.int32, (Tb, EPB), 1)
    neg = jnp.float32(-1e30)
    big = jnp.int32(EPB)

    m = jnp.max(sel, axis=1, keepdims=True)
    S = jnp.sum(jnp.exp(sel - m), axis=1, keepdims=True)

    i1 = jnp.min(jnp.where(sel == m, lane8, big), axis=1, keepdims=True)
    p1 = 1.0 / S

    s2 = jnp.where(lane8 == i1, neg, sel)
    m2 = jnp.max(s2, axis=1, keepdims=True)
    i2 = jnp.min(jnp.where(s2 == m2, lane8, big), axis=1, keepdims=True)
    p2 = jnp.exp(m2 - m) * p1

    denom = p1 + p2 + 1e-9
    base = b * EPB
    gid_ref[...] = jnp.concatenate([base + i1, base + i2], axis=1)
    w_ref[...] = jnp.concatenate([p1 / denom, p2 / denom], axis=1).astype(jnp.float32)


@jax.jit
def _route(h2, b2, keys2):
    T, C = h2.shape
    Tb = 512
    grid = (T // Tb,)
    gid, w = pl.pallas_call(
        _router_block,
        grid=grid,
        in_specs=[
            pl.BlockSpec((Tb, C), lambda i: (i, 0)),
            pl.BlockSpec((Tb, 1), lambda i: (i, 0)),
            pl.BlockSpec((NKEYS, C), lambda i: (0, 0)),
        ],
        out_specs=[
            pl.BlockSpec((Tb, 2), lambda i: (i, 0)),
            pl.BlockSpec((Tb, 2), lambda i: (i, 0)),
        ],
        out_shape=[
            jax.ShapeDtypeStruct((T, 2), jnp.int32),
            jax.ShapeDtypeStruct((T, 2), jnp.float32),
        ],
        scratch_shapes=[pltpu.VMEM((NKEYS, C), jnp.float32)],
    )(h2, b2, keys2)
    return gid, w


def kernel(h, op_id, expert_key):
    B, T, C = h.shape
    h2 = h.reshape(B * T, C)
    b2 = jnp.clip(op_id, 0, N_BUCKET - 1).astype(jnp.int32).reshape(B * T, 1)
    keys2 = expert_key.reshape(N_BUCKET, EPB, C).transpose(1, 0, 2).reshape(NKEYS, C)
    gid, w = _route(h2, b2, keys2)
    return gid.reshape(B, T, 2), w.reshape(B, T, 2)


# speedup vs baseline: 2.4610x; 2.4610x over previous
import jax
import jax.numpy as jnp
from jax.experimental import pallas as pl
from jax.experimental.pallas import tpu as pltpu

N_BUCKET = 64
EPB = 8
NKEYS = N_BUCKET * EPB


def _router_block(h_ref, b_ref, keys_ref, gid_ref, w_ref, kn_ref):
    @pl.when(pl.program_id(0) == 0)
    def _normalize_keys():
        keys = keys_ref[...]
        norm = jnp.sqrt(jnp.sum(keys * keys, axis=1, keepdims=True))
        kn_ref[...] = keys * (1.0 / jnp.maximum(norm, 1e-12))

    h = h_ref[...]
    Tb = h.shape[0]

    scoresT = jax.lax.dot_general(
        kn_ref[...], h, (((1,), (1,)), ((), ())),
        preferred_element_type=jnp.float32,
        precision=jax.lax.Precision.DEFAULT,
    )

    normsq = jnp.sum(h * h, axis=1, keepdims=True)
    rh = 1.0 / jnp.maximum(jnp.sqrt(jnp.transpose(normsq)), 1e-12)

    b = b_ref[...]
    g_iota = jax.lax.broadcasted_iota(jnp.int32, (N_BUCKET, Tb), 0)
    ohT = jnp.where(g_iota == b, 1.0, 0.0)

    selT = jnp.concatenate(
        [
            jnp.sum(scoresT[j * N_BUCKET:(j + 1) * N_BUCKET, :] * ohT,
                    axis=0, keepdims=True)
            for j in range(EPB)
        ],
        axis=0,
    ) * rh

    j_iota = jax.lax.broadcasted_iota(jnp.int32, (EPB, Tb), 0)
    neg = jnp.float32(-1e30)
    big = jnp.int32(EPB)

    m = jnp.max(selT, axis=0, keepdims=True)
    S = jnp.sum(jnp.exp(selT - m), axis=0, keepdims=True)

    i1 = jnp.min(jnp.where(selT == m, j_iota, big), axis=0, keepdims=True)
    p1 = 1.0 / S

    s2 = jnp.where(j_iota == i1, neg, selT)
    m2 = jnp.max(s2, axis=0, keepdims=True)
    i2 = jnp.min(jnp.where(s2 == m2, j_iota, big), axis=0, keepdims=True)
    p2 = jnp.exp(m2 - m) * p1

    denom = p1 + p2 + 1e-9
    base = b * EPB
    gid_ref[...] = jnp.concatenate([base + i1, base + i2], axis=0)
    w_ref[...] = jnp.concatenate([p1 / denom, p2 / denom], axis=0).astype(jnp.float32)


@jax.jit
def _route(h2, bT, keys2):
    T, C = h2.shape
    Tb = 512
    grid = (T // Tb,)
    gidT, wT = pl.pallas_call(
        _router_block,
        grid=grid,
        in_specs=[
            pl.BlockSpec((Tb, C), lambda i: (i, 0)),
            pl.BlockSpec((1, Tb), lambda i: (0, i)),
            pl.BlockSpec((NKEYS, C), lambda i: (0, 0)),
        ],
        out_specs=[
            pl.BlockSpec((2, Tb), lambda i: (0, i)),
            pl.BlockSpec((2, Tb), lambda i: (0, i)),
        ],
        out_shape=[
            jax.ShapeDtypeStruct((2, T), jnp.int32),
            jax.ShapeDtypeStruct((2, T), jnp.float32),
        ],
        scratch_shapes=[pltpu.VMEM((NKEYS, C), jnp.float32)],
    )(h2, bT, keys2)
    return gidT, wT


def kernel(h, op_id, expert_key):
    B, T, C = h.shape
    h2 = h.reshape(B * T, C)
    bT = jnp.clip(op_id, 0, N_BUCKET - 1).astype(jnp.int32).reshape(1, B * T)
    keys2 = expert_key.reshape(N_BUCKET, EPB, C).transpose(1, 0, 2).reshape(NKEYS, C)
    gidT, wT = _route(h2, bT, keys2)
    gid = jnp.transpose(gidT).reshape(B, T, 2)
    w = jnp.transpose(wT).reshape(B, T, 2)
    return gid, w


# speedup vs baseline: 3.0219x; 1.2279x over previous
import jax
import jax.numpy as jnp
from jax.experimental import pallas as pl
from jax.experimental.pallas import tpu as pltpu

N_BUCKET = 64
EPB = 8
NKEYS = N_BUCKET * EPB


def _router_block(h_ref, b_ref, keys_ref, gid_ref, w_ref, kn_ref):
    @pl.when(pl.program_id(0) == 0)
    def _normalize_keys():
        keys = keys_ref[...]
        norm = jnp.sqrt(jnp.sum(keys * keys, axis=1, keepdims=True))
        kn_ref[...] = keys * (1.0 / jnp.maximum(norm, 1e-12))

    h = h_ref[...]
    Tb = h.shape[0]

    scoresT = jax.lax.dot_general(
        kn_ref[...], h, (((1,), (1,)), ((), ())),
        preferred_element_type=jnp.float32,
        precision=jax.lax.Precision.DEFAULT,
    )

    normsq = jnp.sum(h * h, axis=1, keepdims=True)
    rh = 1.0 / jnp.maximum(jnp.sqrt(jnp.transpose(normsq)), 1e-12)

    b = b_ref[...]
    g_iota = jax.lax.broadcasted_iota(jnp.int32, (N_BUCKET, Tb), 0)
    ohT = jnp.where(g_iota == b, 1.0, 0.0)

    selT = jnp.concatenate(
        [
            jnp.sum(scoresT[j * N_BUCKET:(j + 1) * N_BUCKET, :] * ohT,
                    axis=0, keepdims=True)
            for j in range(EPB)
        ],
        axis=0,
    ) * rh

    j_iota = jax.lax.broadcasted_iota(jnp.int32, (EPB, Tb), 0)
    neg = jnp.float32(-1e30)
    big = jnp.int32(EPB)

    m = jnp.max(selT, axis=0, keepdims=True)
    S = jnp.sum(jnp.exp(selT - m), axis=0, keepdims=True)

    i1 = jnp.min(jnp.where(selT == m, j_iota, big), axis=0, keepdims=True)
    p1 = 1.0 / S

    s2 = jnp.where(j_iota == i1, neg, selT)
    m2 = jnp.max(s2, axis=0, keepdims=True)
    i2 = jnp.min(jnp.where(s2 == m2, j_iota, big), axis=0, keepdims=True)
    p2 = jnp.exp(m2 - m) * p1

    denom = p1 + p2 + 1e-9
    base = b * EPB
    gid_ref[...] = jnp.concatenate([base + i1, base + i2], axis=0)
    w_ref[...] = jnp.concatenate([p1 / denom, p2 / denom], axis=0).astype(jnp.float32)


@jax.jit
def _route(h2, bT, keys2):
    T, C = h2.shape
    Tb = 2048
    grid = (T // Tb,)
    gidT, wT = pl.pallas_call(
        _router_block,
        grid=grid,
        in_specs=[
            pl.BlockSpec((Tb, C), lambda i: (i, 0)),
            pl.BlockSpec((1, Tb), lambda i: (0, i)),
            pl.BlockSpec((NKEYS, C), lambda i: (0, 0)),
        ],
        out_specs=[
            pl.BlockSpec((2, Tb), lambda i: (0, i)),
            pl.BlockSpec((2, Tb), lambda i: (0, i)),
        ],
        out_shape=[
            jax.ShapeDtypeStruct((2, T), jnp.int32),
            jax.ShapeDtypeStruct((2, T), jnp.float32),
        ],
        scratch_shapes=[pltpu.VMEM((NKEYS, C), jnp.float32)],
    )(h2, bT, keys2)
    return gidT, wT


def kernel(h, op_id, expert_key):
    B, T, C = h.shape
    h2 = h.reshape(B * T, C)
    bT = jnp.clip(op_id, 0, N_BUCKET - 1).astype(jnp.int32).reshape(1, B * T)
    keys2 = expert_key.reshape(N_BUCKET, EPB, C).transpose(1, 0, 2).reshape(NKEYS, C)
    gidT, wT = _route(h2, bT, keys2)
    gid = jnp.transpose(gidT).reshape(B, T, 2)
    w = jnp.transpose(wT).reshape(B, T, 2)
    return gid, w


# speedup vs baseline: 3.6759x; 1.2164x over previous
import jax
import jax.numpy as jnp
from jax.experimental import pallas as pl
from jax.experimental.pallas import tpu as pltpu

N_BUCKET = 64
EPB = 8
NKEYS = N_BUCKET * EPB


def _router_block(h_ref, b_ref, keys_ref, gid_ref, w_ref, kn_ref):
    @pl.when(pl.program_id(0) == 0)
    def _normalize_keys():
        keys = keys_ref[...]
        norm = jnp.sqrt(jnp.sum(keys * keys, axis=1, keepdims=True))
        kn_ref[...] = keys * (1.0 / jnp.maximum(norm, 1e-12))

    h = h_ref[...]
    Tb = h.shape[0]

    scoresT = jax.lax.dot_general(
        kn_ref[...], h, (((1,), (1,)), ((), ())),
        preferred_element_type=jnp.float32,
        precision=jax.lax.Precision.DEFAULT,
    )

    normsq = jnp.sum(h * h, axis=1, keepdims=True)
    rh = 1.0 / jnp.maximum(jnp.sqrt(jnp.transpose(normsq)), 1e-12)

    b = jnp.clip(b_ref[...], 0, N_BUCKET - 1)
    r_iota = jax.lax.broadcasted_iota(jnp.int32, (NKEYS, Tb), 0)
    masked = jnp.where((r_iota >> 3) == b, scoresT, 0.0)
    selT = jnp.sum(masked.reshape(N_BUCKET, EPB, Tb), axis=0) * rh

    j_iota = jax.lax.broadcasted_iota(jnp.int32, (EPB, Tb), 0)
    neg = jnp.float32(-1e30)
    big = jnp.int32(EPB)

    m = jnp.max(selT, axis=0, keepdims=True)
    S = jnp.sum(jnp.exp(selT - m), axis=0, keepdims=True)

    i1 = jnp.min(jnp.where(selT == m, j_iota, big), axis=0, keepdims=True)
    p1 = 1.0 / S

    s2 = jnp.where(j_iota == i1, neg, selT)
    m2 = jnp.max(s2, axis=0, keepdims=True)
    i2 = jnp.min(jnp.where(s2 == m2, j_iota, big), axis=0, keepdims=True)
    p2 = jnp.exp(m2 - m) * p1

    denom = p1 + p2 + 1e-9
    base = b * EPB
    gid_ref[...] = jnp.concatenate([base + i1, base + i2], axis=0)
    w_ref[...] = jnp.concatenate([p1 / denom, p2 / denom], axis=0).astype(jnp.float32)


@jax.jit
def _route(h2, bT, keys2):
    T, C = h2.shape
    Tb = 2048
    grid = (T // Tb,)
    gidT, wT = pl.pallas_call(
        _router_block,
        grid=grid,
        in_specs=[
            pl.BlockSpec((Tb, C), lambda i: (i, 0)),
            pl.BlockSpec((1, Tb), lambda i: (0, i)),
            pl.BlockSpec((NKEYS, C), lambda i: (0, 0)),
        ],
        out_specs=[
            pl.BlockSpec((2, Tb), lambda i: (0, i)),
            pl.BlockSpec((2, Tb), lambda i: (0, i)),
        ],
        out_shape=[
            jax.ShapeDtypeStruct((2, T), jnp.int32),
            jax.ShapeDtypeStruct((2, T), jnp.float32),
        ],
        scratch_shapes=[pltpu.VMEM((NKEYS, C), jnp.float32)],
    )(h2, bT, keys2)
    return gidT, wT


def kernel(h, op_id, expert_key):
    B, T, C = h.shape
    h2 = h.reshape(B * T, C)
    bT = op_id.astype(jnp.int32).reshape(1, B * T)
    keys2 = expert_key.reshape(NKEYS, C)
    gidT, wT = _route(h2, bT, keys2)
    gid = jnp.transpose(gidT).reshape(B, T, 2)
    w = jnp.transpose(wT).reshape(B, T, 2)
    return gid, w
